# baseline (speedup 1.0000x reference)
import functools

import jax
import jax.numpy as jnp
from jax import lax
from jax.experimental import pallas as pl
from jax.experimental.pallas import tpu as pltpu
from jax.experimental.pallas import tpu_sc as plsc

_FEAT = 64
_HID = 32
_RESO = 64
_PAD = 0.1
_NB = _RESO * _RESO
_GROWS = 4224
_NC, _NS = 2, 16
_NW = _NC * _NS
_CHUNK = 128
_NPAD = 102400
_ROWS_PER_W = _NPAD // _NW
_NCHUNKS = _ROWS_PER_W // _CHUNK
_RB = 2048


def _sc_mesh():
    return plsc.VectorSubcoreMesh(
        core_axis_name="c", subcore_axis_name="s",
        num_cores=_NC, num_subcores=_NS)


def _make_scatter(nplanes, width):
    mesh = _sc_mesh()
    out_type = [jax.ShapeDtypeStruct((_NC, _GROWS, width), jnp.float32)
                for _ in range(nplanes)]
    scratch = ([pltpu.VMEM((_CHUNK,), jnp.int32) for _ in range(nplanes)]
               + [pltpu.VMEM((_CHUNK, width), jnp.float32)]
               + [pltpu.VMEM_SHARED((_GROWS, width), jnp.float32)
                  for _ in range(nplanes)])

    @functools.partial(pl.kernel, out_type=out_type, mesh=mesh,
                       scratch_types=scratch,
                       compiler_params=pltpu.CompilerParams(
                           use_tc_tiling_on_sc=False))
    def k(feats_hbm, idx_hbm, zeros_hbm, *rest):
        outs = rest[:nplanes]
        idx_vs = rest[nplanes:2 * nplanes]
        rows_v = rest[2 * nplanes]
        grids = rest[2 * nplanes + 1:]
        cid = lax.axis_index("c")
        sid = lax.axis_index("s")
        wid = sid * _NC + cid

        rpt = _GROWS // _NS
        for p in range(nplanes):
            pltpu.sync_copy(zeros_hbm.at[pl.ds(sid * rpt, rpt)],
                            grids[p].at[pl.ds(sid * rpt, rpt)])
        plsc.subcore_barrier()

        def body(j, _):
            base = wid * _ROWS_PER_W + j * _CHUNK
            pltpu.sync_copy(feats_hbm.at[pl.ds(base, _CHUNK)], rows_v)
            for p in range(nplanes):
                pltpu.sync_copy(idx_hbm.at[p, wid * _NCHUNKS + j],
                                idx_vs[p])
                pltpu.sync_copy(rows_v, grids[p].at[idx_vs[p]], add=True)
            return 0
        lax.fori_loop(0, _NCHUNKS, body, 0)

        plsc.subcore_barrier()
        for p in range(nplanes):
            pltpu.sync_copy(grids[p].at[pl.ds(sid * rpt, rpt)],
                            outs[p].at[cid, pl.ds(sid * rpt, rpt)])

    return k


@functools.cache
def _get_scatter(nplanes, width):
    return _make_scatter(nplanes, width)


def _scatter1(feats, idx, zeros):
    return _get_scatter(1, _HID + 16)(feats, idx, zeros)


def _scatter3(feats, idx, zeros):
    return _get_scatter(3, _FEAT + 16)(feats, idx, zeros)


def _make_gather_mean():
    mesh = _sc_mesh()
    w = _HID + 16

    @functools.partial(
        pl.kernel, mesh=mesh,
        out_type=jax.ShapeDtypeStruct((_NPAD, _HID), jnp.float32),
        scratch_types=[pltpu.VMEM((_NCHUNKS, _CHUNK), jnp.int32),
                       pltpu.VMEM((2, _CHUNK, _HID), jnp.float32),
                       pltpu.VMEM((_NC, _GROWS // _NS, w), jnp.float32),
                       pltpu.VMEM((_GROWS // _NS, _HID), jnp.float32),
                       pltpu.VMEM_SHARED((_GROWS, _HID), jnp.float32),
                       pltpu.SemaphoreType.DMA,
                       pltpu.SemaphoreType.DMA],
        compiler_params=pltpu.CompilerParams(use_tc_tiling_on_sc=False,
                                             needs_layout_passes=False))
    def k(sums_hbm, idx_hbm, out_hbm, idx_v, o_v, part_v, mean_v, mean_sp,
          gsem, osem):
        cid = lax.axis_index("c")
        sid = lax.axis_index("s")
        wid = sid * _NC + cid
        rpt = _GROWS // _NS

        r0 = sid * rpt
        for c_ in range(_NC):
            pltpu.sync_copy(sums_hbm.at[c_, pl.ds(r0, rpt)], part_v.at[c_])

        def norm_row(r, _):
            cntv = (part_v[0, r, pl.ds(_HID, 16)]
                    + part_v[1, r, pl.ds(_HID, 16)])
            rv = 1.0 / jnp.maximum(jnp.full((16,), cntv[0], jnp.float32),
                                   1.0)
            for h in range(_HID // 16):
                a = (part_v[0, r, pl.ds(h * 16, 16)]
                     + part_v[1, r, pl.ds(h * 16, 16)])
                mean_v[r, pl.ds(h * 16, 16)] = a * rv
            return 0
        lax.fori_loop(0, rpt, norm_row, 0)
        pltpu.sync_copy(mean_v, mean_sp.at[pl.ds(r0, rpt)])
        pltpu.sync_copy(idx_hbm.at[pl.ds(wid * _NCHUNKS, _NCHUNKS)], idx_v)
        plsc.subcore_barrier()

        def fire(j, buf):
            pltpu.async_copy(mean_sp.at[idx_v.at[j]], o_v.at[buf], gsem)

        def wait_g(j, buf):
            pltpu.make_async_copy(mean_sp.at[idx_v.at[j]], o_v.at[buf],
                                  gsem).wait()

        def odst(j):
            return out_hbm.at[pl.ds(wid * _ROWS_PER_W + j * _CHUNK, _CHUNK)]

        fire(0, 0)

        def body(j, _):
            buf = j % 2
            nbuf = (j + 1) % 2
            wait_g(j, buf)
            @pl.when(j >= 1)
            def _():
                pltpu.make_async_copy(o_v.at[nbuf], odst(j - 1),
                                      osem).wait()
            @pl.when(j < _NCHUNKS - 1)
            def _():
                fire(j + 1, nbuf)
            pltpu.async_copy(o_v.at[buf], odst(j), osem)
            return 0
        lax.fori_loop(0, _NCHUNKS, body, 0)

        pltpu.make_async_copy(o_v.at[(_NCHUNKS - 1) % 2],
                              odst(_NCHUNKS - 1), osem).wait()

    return k


@functools.cache
def _get_gather_mean():
    return _make_gather_mean()


def _gather_mean(sums, idx):
    return _get_gather_mean()(sums, idx)


_NGRP = _NPAD // _CHUNK
_QPARTS, _CPARTS = 4, 8
_CPT = _FEAT // _CPARTS
_CSTRIDE = _CPT + 1
_QPT = _NPAD // _QPARTS
_GPT = _QPT // _CHUNK


def _make_sampler():
    mesh = _sc_mesh()

    @functools.partial(
        pl.kernel, mesh=mesh,
        out_type=jax.ShapeDtypeStruct((_FEAT, _NPAD), jnp.float32),
        scratch_types=[pltpu.VMEM((3, _NB * _CSTRIDE), jnp.float32),
                       pltpu.VMEM((512, _CPT), jnp.float32),
                       pltpu.VMEM((2, _CPT, _CHUNK), jnp.float32),
                       pltpu.VMEM((2, 12, _CHUNK), jnp.int32),
                       pltpu.VMEM((2, 12, _CHUNK), jnp.float32),
                       pltpu.SemaphoreType.DMA,
                       pltpu.SemaphoreType.DMA],
        compiler_params=pltpu.CompilerParams(use_tc_tiling_on_sc=False,
                                             needs_layout_passes=False))
    def k(tabs_hbm, idx_hbm, w_hbm, out_hbm,
          tab_v, stage_v, out_g, idx_v, w_v, sem, osem):
        cid = lax.axis_index("c")
        sid = lax.axis_index("s")
        wid = sid * _NC + cid
        qpart = wid // _CPARTS
        cpart = wid % _CPARTS
        c0 = cpart * _CPT
        g0 = qpart * _GPT

        for p in range(3):
            def stage_piece(s, _):
                pltpu.sync_copy(
                    tabs_hbm.at[p, pl.ds(s * 512, 512), pl.ds(c0, _CPT)],
                    stage_v)

                def spread(r, _):
                    rows = lax.iota(jnp.int32, 16) + r * 16
                    for c in range(_CPT):
                        cs = jnp.full((16,), c, jnp.int32)
                        v = plsc.load_gather(stage_v, [rows, cs])
                        plsc.store_scatter(
                            tab_v.at[p],
                            [(rows + s * 512) * _CSTRIDE + c], v)
                    return 0
                lax.fori_loop(0, 32, spread, 0)
                return 0
            lax.fori_loop(0, _NB // 512, stage_piece, 0)

        def issue(g, buf):
            pltpu.async_copy(idx_hbm.at[g0 + g], idx_v.at[buf], sem)
            pltpu.async_copy(w_hbm.at[g0 + g], w_v.at[buf], sem)

        def wait_in(g, buf):
            pltpu.make_async_copy(idx_hbm.at[g0 + g], idx_v.at[buf],
                                  sem).wait()
            pltpu.make_async_copy(w_hbm.at[g0 + g], w_v.at[buf],
                                  sem).wait()

        def out_dst(g, buf):
            qstart = qpart * _QPT + g * _CHUNK
            return out_hbm.at[pl.ds(c0, _CPT), pl.ds(qstart, _CHUNK)]

        issue(0, 0)

        def group_body(g, _):
            buf = g % 2
            @pl.when(g < _GPT - 1)
            def _():
                issue(g + 1, (g + 1) % 2)
            wait_in(g, buf)
            @pl.when(g >= 2)
            def _():
                pltpu.make_async_copy(out_g.at[buf], out_dst(g - 2, buf),
                                      osem).wait()
            cur_i = idx_v.at[buf]
            cur_w = w_v.at[buf]
            for qb in range(_CHUNK // 16):
                qi = [cur_i[t, pl.ds(qb * 16, 16)] * _CSTRIDE
                      for t in range(12)]
                wv = [cur_w[t, pl.ds(qb * 16, 16)] for t in range(12)]
                for cb in range(_CPT // 4):
                    cs = [cb * 4 + i for i in range(4)]
                    accs = [wv[0] * plsc.load_gather(tab_v.at[0],
                                                     [qi[0] + c])
                            for c in cs]
                    for t in range(1, 12):
                        gs = [plsc.load_gather(tab_v.at[t // 4],
                                               [qi[t] + c]) for c in cs]
                        accs = [a + wv[t] * g for a, g in zip(accs, gs)]
                    for c, a in zip(cs, accs):
                        out_g[buf, c, pl.ds(qb * 16, 16)] = a
            pltpu.async_copy(out_g.at[buf], out_dst(g, buf), osem)
            return 0
        lax.fori_loop(0, _GPT, group_body, 0)

        pltpu.make_async_copy(out_g.at[_GPT % 2],
                              out_dst(_GPT - 2, _GPT % 2), osem).wait()
        pltpu.make_async_copy(out_g.at[(_GPT - 1) % 2],
                              out_dst(_GPT - 1, (_GPT - 1) % 2),
                              osem).wait()

    return k


@functools.cache
def _get_sampler():
    return _make_sampler()


def _sample(tabs, tapidx, tapw):
    return _get_sampler()(tabs, tapidx, tapw)


_GRB = _RB // _CHUNK

_PLANE_DIMS = ((0, 2), (0, 1), (1, 2))


def _stage_a_body(p_ref, pr_ref, qr_ref, w8_ref, b_ref, w0_ref, b0_ref,
                  w1_ref, b1_ref, ws_ref, o_ref, oi_ref, oti_ref, otw_ref,
                  *, n):
    x = jnp.dot(p_ref[...], w8_ref[...], preferred_element_type=jnp.float32)
    x = x + b_ref[...]
    h = jax.nn.relu(jnp.dot(x, w0_ref[...], preferred_element_type=jnp.float32)
                    + b0_ref[...])
    dx = jax.nn.relu(jnp.dot(h, w1_ref[...], preferred_element_type=jnp.float32)
                     + b1_ref[...])
    net = jnp.dot(x, ws_ref[...], preferred_element_type=jnp.float32) + dx
    ones = jnp.ones((_RB, 1), jnp.float32)
    zeros = jnp.zeros((_RB, 15), jnp.float32)
    o_ref[...] = jnp.concatenate([net, ones, zeros], axis=1)

    base = pl.program_id(0) * _RB
    gr = (base + lax.broadcasted_iota(jnp.int32, (_GRB, _CHUNK), 0) * _CHUNK
          + lax.broadcasted_iota(jnp.int32, (_GRB, _CHUNK), 1))
    for d_i, d in enumerate(_PLANE_DIMS):
        xy0 = jnp.clip((pr_ref[:, :, d[0]] + 0.5) / (1.0 + _PAD + 1e-6),
                       0.0, 1.0 - 1e-6)
        xy1 = jnp.clip((pr_ref[:, :, d[1]] + 0.5) / (1.0 + _PAD + 1e-6),
                       0.0, 1.0 - 1e-6)
        xi0 = jnp.floor(xy0 * _RESO).astype(jnp.int32)
        xi1 = jnp.floor(xy1 * _RESO).astype(jnp.int32)
        oi_ref[d_i] = jnp.where(gr < n, xi0 + _RESO * xi1, _NB)

    for d_i, d in enumerate(_PLANE_DIMS):
        xq = qr_ref[:, :, d[0]] * _RESO - 0.5
        yq = qr_ref[:, :, d[1]] * _RESO - 0.5
        x0 = jnp.floor(xq)
        y0 = jnp.floor(yq)
        wx = xq - x0
        wy = yq - y0
        x0i = jnp.clip(x0, 0, _RESO - 1).astype(jnp.int32)
        x1i = jnp.clip(x0 + 1.0, 0, _RESO - 1).astype(jnp.int32)
        y0i = jnp.clip(y0, 0, _RESO - 1).astype(jnp.int32)
        y1i = jnp.clip(y0 + 1.0, 0, _RESO - 1).astype(jnp.int32)
        taps = ((y0i, x0i, (1 - wx) * (1 - wy)),
                (y0i, x1i, wx * (1 - wy)),
                (y1i, x0i, (1 - wx) * wy),
                (y1i, x1i, wx * wy))
        for t, (yi, xi, wt) in enumerate(taps):
            oti_ref[:, 4 * d_i + t] = yi * _RESO + xi
            otw_ref[:, 4 * d_i + t] = wt


def _run_stage_a(p8, p_r, q_r, w8, b, w0, b0, w1, b1, ws, n):
    grid = _NPAD // _RB
    full = lambda shp: pl.BlockSpec(shp, lambda i: (0, 0))
    return pl.pallas_call(
        functools.partial(_stage_a_body, n=n),
        grid=(grid,),
        in_specs=[pl.BlockSpec((_RB, 8), lambda i: (i, 0)),
                  pl.BlockSpec((_GRB, _CHUNK, 3), lambda i: (i, 0, 0)),
                  pl.BlockSpec((_GRB, _CHUNK, 3), lambda i: (i, 0, 0)),
                  full((8, 2 * _HID)), full((1, 2 * _HID)),
                  full((2 * _HID, _HID)), full((1, _HID)),
                  full((_HID, _HID)), full((1, _HID)),
                  full((2 * _HID, _HID))],
        out_specs=[pl.BlockSpec((_RB, _HID + 16), lambda i: (i, 0)),
                   pl.BlockSpec((3, _GRB, _CHUNK), lambda i: (0, i, 0)),
                   pl.BlockSpec((_GRB, 12, _CHUNK), lambda i: (i, 0, 0)),
                   pl.BlockSpec((_GRB, 12, _CHUNK), lambda i: (i, 0, 0))],
        out_shape=[jax.ShapeDtypeStruct((_NPAD, _HID + 16), jnp.float32),
                   jax.ShapeDtypeStruct((3, _NGRP, _CHUNK), jnp.int32),
                   jax.ShapeDtypeStruct((_NGRP, 12, _CHUNK), jnp.int32),
                   jax.ShapeDtypeStruct((_NGRP, 12, _CHUNK), jnp.float32)],
    )(p8, p_r, q_r, w8, b, w0, b0, w1, b1, ws)


def _block_body(net_ref, pool_ref, w0_ref, b0_ref, w1_ref, b1_ref, ws_ref,
                fcw_ref, fcb_ref, o_ref, *, out_width, with_fc):
    net = net_ref[:, :_HID]
    pool = pool_ref[...]
    w0 = w0_ref[...]
    ws = ws_ref[...]
    xw0 = (jnp.dot(net, w0[:_HID], preferred_element_type=jnp.float32)
           + jnp.dot(pool, w0[_HID:], preferred_element_type=jnp.float32))
    h = jax.nn.relu(xw0 + b0_ref[...])
    dx = jax.nn.relu(jnp.dot(h, w1_ref[...], preferred_element_type=jnp.float32)
                     + b1_ref[...])
    out = (jnp.dot(net, ws[:_HID], preferred_element_type=jnp.float32)
           + jnp.dot(pool, ws[_HID:], preferred_element_type=jnp.float32)
           + dx)
    if with_fc:
        out = jnp.dot(out, fcw_ref[...], preferred_element_type=jnp.float32) \
            + fcb_ref[...]
        width = _FEAT
    else:
        width = _HID
    ones = jnp.ones((_RB, 1), jnp.float32)
    zeros = jnp.zeros((_RB, out_width - width - 1), jnp.float32)
    o_ref[...] = jnp.concatenate([out, ones, zeros], axis=1)


def _run_block(net, pool, w0, b0, w1, b1, ws, fcw, fcb, with_fc):
    out_width = (_FEAT + 16) if with_fc else (_HID + 16)
    grid = _NPAD // _RB
    full = lambda shp: pl.BlockSpec(shp, lambda i: (0, 0))
    body = functools.partial(_block_body, out_width=out_width, with_fc=with_fc)
    return pl.pallas_call(
        body,
        grid=(grid,),
        in_specs=[pl.BlockSpec((_RB, _HID + 16), lambda i: (i, 0)),
                  pl.BlockSpec((_RB, _HID), lambda i: (i, 0)),
                  full((2 * _HID, _HID)), full((1, _HID)),
                  full((_HID, _HID)), full((1, _HID)),
                  full((2 * _HID, _HID)),
                  full((_HID, _FEAT)), full((1, _FEAT))],
        out_specs=pl.BlockSpec((_RB, out_width), lambda i: (i, 0)),
        out_shape=jax.ShapeDtypeStruct((_NPAD, out_width), jnp.float32),
    )(net, pool, w0, b0, w1, b1, ws, fcw, fcb)


def _conv_taps(x, wk, bias, col):
    acc = jnp.zeros((_NB, _FEAT), jnp.float32)
    xp = jnp.concatenate([jnp.zeros((_RESO + 1, _FEAT), jnp.float32), x,
                          jnp.zeros((_RESO + 1, _FEAT), jnp.float32)], axis=0)
    k = 0
    for di in (-1, 0, 1):
        for dj in (-1, 0, 1):
            s = di * _RESO + dj
            tap = lax.slice_in_dim(xp, _RESO + 1 + s, _RESO + 1 + s + _NB,
                                   axis=0)
            if dj == -1:
                tap = jnp.where(col >= 1, tap, 0.0)
            elif dj == 1:
                tap = jnp.where(col <= _RESO - 2, tap, 0.0)
            acc = acc + jnp.dot(tap, wk[k],
                                preferred_element_type=jnp.float32)
            k += 1
    return jax.nn.relu(acc + bias)


def _conv_body(s_ref, w1_ref, b1_ref, w2_ref, b2_ref, o_ref):
    s = s_ref[0, 0] + s_ref[0, 1]
    s = s[:_NB]
    mean = s[:, :_FEAT] / jnp.maximum(s[:, _FEAT:_FEAT + 1], 1.0)
    col = lax.broadcasted_iota(jnp.int32, (_NB, _FEAT), 0) % _RESO
    h = _conv_taps(mean, w1_ref, b1_ref[...], col)
    o_ref[0] = _conv_taps(h, w2_ref, b2_ref[...], col)


def _run_convs(sums3, w1k, b1, w2k, b2):
    full = lambda shp: pl.BlockSpec(shp, lambda i: tuple(0 for _ in shp))
    return pl.pallas_call(
        _conv_body,
        grid=(3,),
        in_specs=[pl.BlockSpec((1, _NC, _GROWS, _FEAT + 16),
                               lambda i: (i, 0, 0, 0)),
                  full((9, _FEAT, _FEAT)), full((1, _FEAT)),
                  full((9, _FEAT, _FEAT)), full((1, _FEAT))],
        out_specs=pl.BlockSpec((1, _NB, _FEAT), lambda i: (i, 0, 0)),
        out_shape=jax.ShapeDtypeStruct((3, _NB, _FEAT), jnp.float32),
    )(sums3, w1k, b1, w2k, b2)


def kernel(p, query, params):
    n = p.shape[1]
    m = query.shape[1]
    p2 = p[0]
    q2 = query[0]

    pad_n = _NPAD - n
    pad_m = _NPAD - m
    p8 = jnp.pad(p2, ((0, pad_n), (0, 5)))
    p_r = p8[:, :3].reshape(_NGRP, _CHUNK, 3)
    q_r = jnp.pad(q2, ((0, pad_m), (0, 0))).reshape(_NGRP, _CHUNK, 3)

    pr = params
    w8 = jnp.zeros((8, 2 * _HID), jnp.float32).at[:3].set(pr["fc_pos_W"])

    def r1(x):
        return x.reshape(1, -1)

    def convk(w):
        return jnp.transpose(w, (2, 3, 1, 0)).reshape(9, _FEAT, _FEAT)

    zeros48 = jnp.zeros((_GROWS, _HID + 16), jnp.float32)
    zeros80 = jnp.zeros((_GROWS, _FEAT + 16), jnp.float32)

    net, idx3, tapidx, tapw = _run_stage_a(
        p8, p_r, q_r, w8, r1(pr["fc_pos_b"]),
        pr["blk0_W0"], r1(pr["blk0_b0"]),
        pr["blk0_W1"], r1(pr["blk0_b1"]), pr["blk0_Ws"], n)
    idx_xz = idx3[:1]

    for i in (1, 2):
        sums = _scatter1(net, idx_xz, zeros48)[0]
        pooled = _gather_mean(sums, idx_xz[0])
        with_fc = i == 2
        net = _run_block(net, pooled,
                         pr["blk%d_W0" % i], r1(pr["blk%d_b0" % i]),
                         pr["blk%d_W1" % i], r1(pr["blk%d_b1" % i]),
                         pr["blk%d_Ws" % i],
                         pr["fc_c_W"], r1(pr["fc_c_b"]), with_fc)

    s0, s1, s2 = _scatter3(net, idx3, zeros80)
    sums3 = jnp.stack([s0, s1, s2])
    tabs = _run_convs(sums3, convk(pr["conv1_W"]), r1(pr["conv1_b"]),
                      convk(pr["conv2_W"]), r1(pr["conv2_b"]))

    out_cm = _sample(tabs, tapidx, tapw)
    return jnp.transpose(out_cm[:, :m])[None]


# speedup vs baseline: 1.5184x; 1.5184x over previous
import functools

import jax
import jax.numpy as jnp
from jax import lax
from jax.experimental import pallas as pl
from jax.experimental.pallas import tpu as pltpu
from jax.experimental.pallas import tpu_sc as plsc

_FEAT = 64
_HID = 32
_RESO = 64
_PAD = 0.1
_NB = _RESO * _RESO
_GROWS = 4224
_NC, _NS = 2, 16
_NW = _NC * _NS
_CHUNK = 128
_NPAD = 102400
_ROWS_PER_W = _NPAD // _NW
_NCHUNKS = _ROWS_PER_W // _CHUNK
_RB = 2048


def _sc_mesh():
    return plsc.VectorSubcoreMesh(
        core_axis_name="c", subcore_axis_name="s",
        num_cores=_NC, num_subcores=_NS)


def _make_scatter(nplanes, width):
    mesh = _sc_mesh()
    out_type = [jax.ShapeDtypeStruct((_NC, _GROWS, width), jnp.float32)
                for _ in range(nplanes)]
    scratch = ([pltpu.VMEM((_CHUNK,), jnp.int32) for _ in range(nplanes)]
               + [pltpu.VMEM((_CHUNK, width), jnp.float32)]
               + [pltpu.VMEM_SHARED((_GROWS, width), jnp.float32)
                  for _ in range(nplanes)])

    @functools.partial(pl.kernel, out_type=out_type, mesh=mesh,
                       scratch_types=scratch,
                       compiler_params=pltpu.CompilerParams(
                           use_tc_tiling_on_sc=False))
    def k(feats_hbm, idx_hbm, zeros_hbm, *rest):
        outs = rest[:nplanes]
        idx_vs = rest[nplanes:2 * nplanes]
        rows_v = rest[2 * nplanes]
        grids = rest[2 * nplanes + 1:]
        cid = lax.axis_index("c")
        sid = lax.axis_index("s")
        wid = sid * _NC + cid

        rpt = _GROWS // _NS
        for p in range(nplanes):
            pltpu.sync_copy(zeros_hbm.at[pl.ds(sid * rpt, rpt)],
                            grids[p].at[pl.ds(sid * rpt, rpt)])
        plsc.subcore_barrier()

        def body(j, _):
            base = wid * _ROWS_PER_W + j * _CHUNK
            pltpu.sync_copy(feats_hbm.at[pl.ds(base, _CHUNK)], rows_v)
            for p in range(nplanes):
                pltpu.sync_copy(idx_hbm.at[p, wid * _NCHUNKS + j],
                                idx_vs[p])
                pltpu.sync_copy(rows_v, grids[p].at[idx_vs[p]], add=True)
            return 0
        lax.fori_loop(0, _NCHUNKS, body, 0)

        plsc.subcore_barrier()
        for p in range(nplanes):
            pltpu.sync_copy(grids[p].at[pl.ds(sid * rpt, rpt)],
                            outs[p].at[cid, pl.ds(sid * rpt, rpt)])

    return k


@functools.cache
def _get_scatter(nplanes, width):
    return _make_scatter(nplanes, width)


def _scatter1(feats, idx, zeros):
    return _get_scatter(1, _HID + 16)(feats, idx, zeros)


def _scatter3(feats, idx, zeros):
    return _get_scatter(3, _FEAT + 16)(feats, idx, zeros)


def _make_gather_mean():
    mesh = _sc_mesh()
    w = _HID + 16

    @functools.partial(
        pl.kernel, mesh=mesh,
        out_type=jax.ShapeDtypeStruct((_NPAD, _HID), jnp.float32),
        scratch_types=[pltpu.VMEM((_NCHUNKS, _CHUNK), jnp.int32),
                       pltpu.VMEM((2, _CHUNK, _HID), jnp.float32),
                       pltpu.VMEM((_NC, _GROWS // _NS, w), jnp.float32),
                       pltpu.VMEM((_GROWS // _NS, _HID), jnp.float32),
                       pltpu.VMEM_SHARED((_GROWS, _HID), jnp.float32),
                       pltpu.SemaphoreType.DMA,
                       pltpu.SemaphoreType.DMA],
        compiler_params=pltpu.CompilerParams(use_tc_tiling_on_sc=False,
                                             needs_layout_passes=False))
    def k(sums_hbm, idx_hbm, out_hbm, idx_v, o_v, part_v, mean_v, mean_sp,
          gsem, osem):
        cid = lax.axis_index("c")
        sid = lax.axis_index("s")
        wid = sid * _NC + cid
        rpt = _GROWS // _NS

        r0 = sid * rpt
        for c_ in range(_NC):
            pltpu.sync_copy(sums_hbm.at[c_, pl.ds(r0, rpt)], part_v.at[c_])

        def norm_row(r, _):
            cntv = (part_v[0, r, pl.ds(_HID, 16)]
                    + part_v[1, r, pl.ds(_HID, 16)])
            rv = 1.0 / jnp.maximum(jnp.full((16,), cntv[0], jnp.float32),
                                   1.0)
            for h in range(_HID // 16):
                a = (part_v[0, r, pl.ds(h * 16, 16)]
                     + part_v[1, r, pl.ds(h * 16, 16)])
                mean_v[r, pl.ds(h * 16, 16)] = a * rv
            return 0
        lax.fori_loop(0, rpt, norm_row, 0)
        pltpu.sync_copy(mean_v, mean_sp.at[pl.ds(r0, rpt)])
        pltpu.sync_copy(idx_hbm.at[pl.ds(wid * _NCHUNKS, _NCHUNKS)], idx_v)
        plsc.subcore_barrier()

        def fire(j, buf):
            pltpu.async_copy(mean_sp.at[idx_v.at[j]], o_v.at[buf], gsem)

        def wait_g(j, buf):
            pltpu.make_async_copy(mean_sp.at[idx_v.at[j]], o_v.at[buf],
                                  gsem).wait()

        def odst(j):
            return out_hbm.at[pl.ds(wid * _ROWS_PER_W + j * _CHUNK, _CHUNK)]

        fire(0, 0)

        def body(j, _):
            buf = j % 2
            nbuf = (j + 1) % 2
            wait_g(j, buf)
            @pl.when(j >= 1)
            def _():
                pltpu.make_async_copy(o_v.at[nbuf], odst(j - 1),
                                      osem).wait()
            @pl.when(j < _NCHUNKS - 1)
            def _():
                fire(j + 1, nbuf)
            pltpu.async_copy(o_v.at[buf], odst(j), osem)
            return 0
        lax.fori_loop(0, _NCHUNKS, body, 0)

        pltpu.make_async_copy(o_v.at[(_NCHUNKS - 1) % 2],
                              odst(_NCHUNKS - 1), osem).wait()

    return k


@functools.cache
def _get_gather_mean():
    return _make_gather_mean()


def _gather_mean(sums, idx):
    return _get_gather_mean()(sums, idx)


_NGRP = _NPAD // _CHUNK
_QPARTS, _CPARTS = 4, 8
_CPT = _FEAT // _CPARTS
_CSTRIDE = _CPT + 1
_QPT = _NPAD // _QPARTS
_GPT = _QPT // _CHUNK


def _make_sampler():
    mesh = _sc_mesh()

    @functools.partial(
        pl.kernel, mesh=mesh,
        out_type=jax.ShapeDtypeStruct((_FEAT, _NPAD), jnp.float32),
        scratch_types=[pltpu.VMEM((3, _NB * _CSTRIDE), jnp.float32),
                       pltpu.VMEM((512, _CPT), jnp.float32),
                       pltpu.VMEM((2, _CPT, _CHUNK), jnp.float32),
                       pltpu.VMEM((2, 12, _CHUNK), jnp.int32),
                       pltpu.VMEM((2, 12, _CHUNK), jnp.float32),
                       pltpu.SemaphoreType.DMA,
                       pltpu.SemaphoreType.DMA],
        compiler_params=pltpu.CompilerParams(use_tc_tiling_on_sc=False,
                                             needs_layout_passes=False))
    def k(tabs_hbm, idx_hbm, w_hbm, out_hbm,
          tab_v, stage_v, out_g, idx_v, w_v, sem, osem):
        cid = lax.axis_index("c")
        sid = lax.axis_index("s")
        wid = sid * _NC + cid
        qpart = wid // _CPARTS
        cpart = wid % _CPARTS
        c0 = cpart * _CPT
        g0 = qpart * _GPT

        for p in range(3):
            def stage_piece(s, _):
                pltpu.sync_copy(
                    tabs_hbm.at[p, pl.ds(s * 512, 512), pl.ds(c0, _CPT)],
                    stage_v)

                def spread(r, _):
                    rows = lax.iota(jnp.int32, 16) + r * 16
                    for c in range(_CPT):
                        cs = jnp.full((16,), c, jnp.int32)
                        v = plsc.load_gather(stage_v, [rows, cs])
                        plsc.store_scatter(
                            tab_v.at[p],
                            [(rows + s * 512) * _CSTRIDE + c], v)
                    return 0
                lax.fori_loop(0, 32, spread, 0)
                return 0
            lax.fori_loop(0, _NB // 512, stage_piece, 0)

        def issue(g, buf):
            pltpu.async_copy(idx_hbm.at[g0 + g], idx_v.at[buf], sem)
            pltpu.async_copy(w_hbm.at[g0 + g], w_v.at[buf], sem)

        def wait_in(g, buf):
            pltpu.make_async_copy(idx_hbm.at[g0 + g], idx_v.at[buf],
                                  sem).wait()
            pltpu.make_async_copy(w_hbm.at[g0 + g], w_v.at[buf],
                                  sem).wait()

        def out_dst(g, buf):
            qstart = qpart * _QPT + g * _CHUNK
            return out_hbm.at[pl.ds(c0, _CPT), pl.ds(qstart, _CHUNK)]

        issue(0, 0)

        def group_body(g, _):
            buf = g % 2
            @pl.when(g < _GPT - 1)
            def _():
                issue(g + 1, (g + 1) % 2)
            wait_in(g, buf)
            @pl.when(g >= 2)
            def _():
                pltpu.make_async_copy(out_g.at[buf], out_dst(g - 2, buf),
                                      osem).wait()
            cur_i = idx_v.at[buf]
            cur_w = w_v.at[buf]
            for qb in range(_CHUNK // 16):
                qi = [cur_i[t, pl.ds(qb * 16, 16)] * _CSTRIDE
                      for t in range(12)]
                wv = [cur_w[t, pl.ds(qb * 16, 16)] for t in range(12)]
                for cb in range(_CPT // 4):
                    cs = [cb * 4 + i for i in range(4)]
                    accs = [wv[0] * plsc.load_gather(tab_v.at[0],
                                                     [qi[0] + c])
                            for c in cs]
                    for t in range(1, 12):
                        gs = [plsc.load_gather(tab_v.at[t // 4],
                                               [qi[t] + c]) for c in cs]
                        accs = [a + wv[t] * g for a, g in zip(accs, gs)]
                    for c, a in zip(cs, accs):
                        out_g[buf, c, pl.ds(qb * 16, 16)] = a
            pltpu.async_copy(out_g.at[buf], out_dst(g, buf), osem)
            return 0
        lax.fori_loop(0, _GPT, group_body, 0)

        pltpu.make_async_copy(out_g.at[_GPT % 2],
                              out_dst(_GPT - 2, _GPT % 2), osem).wait()
        pltpu.make_async_copy(out_g.at[(_GPT - 1) % 2],
                              out_dst(_GPT - 1, (_GPT - 1) % 2),
                              osem).wait()

    return k


@functools.cache
def _get_sampler():
    return _make_sampler()


def _sample(tabs, tapidx, tapw):
    return _get_sampler()(tabs, tapidx, tapw)


_GRB = _RB // _CHUNK

_PLANE_DIMS = ((0, 2), (0, 1), (1, 2))


def _stage_a_body(p_ref, pr_ref, qr_ref, w8_ref, b_ref, w0_ref, b0_ref,
                  w1_ref, b1_ref, ws_ref, o_ref, oi_ref, oti_ref, otw_ref,
                  *, n):
    x = jnp.dot(p_ref[...], w8_ref[...], preferred_element_type=jnp.float32)
    x = x + b_ref[...]
    h = jax.nn.relu(jnp.dot(x, w0_ref[...], preferred_element_type=jnp.float32)
                    + b0_ref[...])
    dx = jax.nn.relu(jnp.dot(h, w1_ref[...], preferred_element_type=jnp.float32)
                     + b1_ref[...])
    net = jnp.dot(x, ws_ref[...], preferred_element_type=jnp.float32) + dx
    ones = jnp.ones((_RB, 1), jnp.float32)
    zeros = jnp.zeros((_RB, 15), jnp.float32)
    o_ref[...] = jnp.concatenate([net, ones, zeros], axis=1)

    base = pl.program_id(0) * _RB
    gr = (base + lax.broadcasted_iota(jnp.int32, (_GRB, _CHUNK), 0) * _CHUNK
          + lax.broadcasted_iota(jnp.int32, (_GRB, _CHUNK), 1))
    pc = [jnp.clip((pr_ref[d] + 0.5) / (1.0 + _PAD + 1e-6),
                   0.0, 1.0 - 1e-6) for d in range(3)]
    for d_i, d in enumerate(_PLANE_DIMS):
        xi0 = jnp.floor(pc[d[0]] * _RESO).astype(jnp.int32)
        xi1 = jnp.floor(pc[d[1]] * _RESO).astype(jnp.int32)
        oi_ref[d_i] = jnp.where(gr < n, xi0 + _RESO * xi1, _NB)

    for d_i, d in enumerate(_PLANE_DIMS):
        xq = qr_ref[d[0]] * _RESO - 0.5
        yq = qr_ref[d[1]] * _RESO - 0.5
        x0 = jnp.floor(xq)
        y0 = jnp.floor(yq)
        wx = xq - x0
        wy = yq - y0
        x0i = jnp.clip(x0, 0, _RESO - 1).astype(jnp.int32)
        x1i = jnp.clip(x0 + 1.0, 0, _RESO - 1).astype(jnp.int32)
        y0i = jnp.clip(y0, 0, _RESO - 1).astype(jnp.int32)
        y1i = jnp.clip(y0 + 1.0, 0, _RESO - 1).astype(jnp.int32)
        taps = ((y0i, x0i, (1 - wx) * (1 - wy)),
                (y0i, x1i, wx * (1 - wy)),
                (y1i, x0i, (1 - wx) * wy),
                (y1i, x1i, wx * wy))
        for t, (yi, xi, wt) in enumerate(taps):
            oti_ref[:, 4 * d_i + t] = yi * _RESO + xi
            otw_ref[:, 4 * d_i + t] = wt


def _run_stage_a(p8, p_r, q_r, w8, b, w0, b0, w1, b1, ws, n):
    grid = _NPAD // _RB
    full = lambda shp: pl.BlockSpec(shp, lambda i: (0, 0))
    return pl.pallas_call(
        functools.partial(_stage_a_body, n=n),
        grid=(grid,),
        in_specs=[pl.BlockSpec((_RB, 8), lambda i: (i, 0)),
                  pl.BlockSpec((3, _GRB, _CHUNK), lambda i: (0, i, 0)),
                  pl.BlockSpec((3, _GRB, _CHUNK), lambda i: (0, i, 0)),
                  full((8, 2 * _HID)), full((1, 2 * _HID)),
                  full((2 * _HID, _HID)), full((1, _HID)),
                  full((_HID, _HID)), full((1, _HID)),
                  full((2 * _HID, _HID))],
        out_specs=[pl.BlockSpec((_RB, _HID + 16), lambda i: (i, 0)),
                   pl.BlockSpec((3, _GRB, _CHUNK), lambda i: (0, i, 0)),
                   pl.BlockSpec((_GRB, 12, _CHUNK), lambda i: (i, 0, 0)),
                   pl.BlockSpec((_GRB, 12, _CHUNK), lambda i: (i, 0, 0))],
        out_shape=[jax.ShapeDtypeStruct((_NPAD, _HID + 16), jnp.float32),
                   jax.ShapeDtypeStruct((3, _NGRP, _CHUNK), jnp.int32),
                   jax.ShapeDtypeStruct((_NGRP, 12, _CHUNK), jnp.int32),
                   jax.ShapeDtypeStruct((_NGRP, 12, _CHUNK), jnp.float32)],
    )(p8, p_r, q_r, w8, b, w0, b0, w1, b1, ws)


def _block_body(net_ref, pool_ref, w0_ref, b0_ref, w1_ref, b1_ref, ws_ref,
                fcw_ref, fcb_ref, o_ref, *, out_width, with_fc):
    net = net_ref[:, :_HID]
    pool = pool_ref[...]
    w0 = w0_ref[...]
    ws = ws_ref[...]
    xw0 = (jnp.dot(net, w0[:_HID], preferred_element_type=jnp.float32)
           + jnp.dot(pool, w0[_HID:], preferred_element_type=jnp.float32))
    h = jax.nn.relu(xw0 + b0_ref[...])
    dx = jax.nn.relu(jnp.dot(h, w1_ref[...], preferred_element_type=jnp.float32)
                     + b1_ref[...])
    out = (jnp.dot(net, ws[:_HID], preferred_element_type=jnp.float32)
           + jnp.dot(pool, ws[_HID:], preferred_element_type=jnp.float32)
           + dx)
    if with_fc:
        out = jnp.dot(out, fcw_ref[...], preferred_element_type=jnp.float32) \
            + fcb_ref[...]
        width = _FEAT
    else:
        width = _HID
    ones = jnp.ones((_RB, 1), jnp.float32)
    zeros = jnp.zeros((_RB, out_width - width - 1), jnp.float32)
    o_ref[...] = jnp.concatenate([out, ones, zeros], axis=1)


def _run_block(net, pool, w0, b0, w1, b1, ws, fcw, fcb, with_fc):
    out_width = (_FEAT + 16) if with_fc else (_HID + 16)
    grid = _NPAD // _RB
    full = lambda shp: pl.BlockSpec(shp, lambda i: (0, 0))
    body = functools.partial(_block_body, out_width=out_width, with_fc=with_fc)
    return pl.pallas_call(
        body,
        grid=(grid,),
        in_specs=[pl.BlockSpec((_RB, _HID + 16), lambda i: (i, 0)),
                  pl.BlockSpec((_RB, _HID), lambda i: (i, 0)),
                  full((2 * _HID, _HID)), full((1, _HID)),
                  full((_HID, _HID)), full((1, _HID)),
                  full((2 * _HID, _HID)),
                  full((_HID, _FEAT)), full((1, _FEAT))],
        out_specs=pl.BlockSpec((_RB, out_width), lambda i: (i, 0)),
        out_shape=jax.ShapeDtypeStruct((_NPAD, out_width), jnp.float32),
    )(net, pool, w0, b0, w1, b1, ws, fcw, fcb)


def _conv_taps(x, wk, bias, col):
    acc = jnp.zeros((_NB, _FEAT), jnp.float32)
    xp = jnp.concatenate([jnp.zeros((_RESO + 1, _FEAT), jnp.float32), x,
                          jnp.zeros((_RESO + 1, _FEAT), jnp.float32)], axis=0)
    k = 0
    for di in (-1, 0, 1):
        for dj in (-1, 0, 1):
            s = di * _RESO + dj
            tap = lax.slice_in_dim(xp, _RESO + 1 + s, _RESO + 1 + s + _NB,
                                   axis=0)
            if dj == -1:
                tap = jnp.where(col >= 1, tap, 0.0)
            elif dj == 1:
                tap = jnp.where(col <= _RESO - 2, tap, 0.0)
            acc = acc + jnp.dot(tap, wk[k],
                                preferred_element_type=jnp.float32)
            k += 1
    return jax.nn.relu(acc + bias)


def _conv_body(s_ref, w1_ref, b1_ref, w2_ref, b2_ref, o_ref):
    s = s_ref[0, 0] + s_ref[0, 1]
    s = s[:_NB]
    mean = s[:, :_FEAT] / jnp.maximum(s[:, _FEAT:_FEAT + 1], 1.0)
    col = lax.broadcasted_iota(jnp.int32, (_NB, _FEAT), 0) % _RESO
    h = _conv_taps(mean, w1_ref, b1_ref[...], col)
    o_ref[0] = _conv_taps(h, w2_ref, b2_ref[...], col)


def _run_convs(sums3, w1k, b1, w2k, b2):
    full = lambda shp: pl.BlockSpec(shp, lambda i: tuple(0 for _ in shp))
    return pl.pallas_call(
        _conv_body,
        grid=(3,),
        in_specs=[pl.BlockSpec((1, _NC, _GROWS, _FEAT + 16),
                               lambda i: (i, 0, 0, 0)),
                  full((9, _FEAT, _FEAT)), full((1, _FEAT)),
                  full((9, _FEAT, _FEAT)), full((1, _FEAT))],
        out_specs=pl.BlockSpec((1, _NB, _FEAT), lambda i: (i, 0, 0)),
        out_shape=jax.ShapeDtypeStruct((3, _NB, _FEAT), jnp.float32),
    )(sums3, w1k, b1, w2k, b2)


def kernel(p, query, params):
    n = p.shape[1]
    m = query.shape[1]
    p2 = p[0]
    q2 = query[0]

    pad_n = _NPAD - n
    pad_m = _NPAD - m
    p8 = jnp.pad(p2, ((0, pad_n), (0, 5)))
    p_r = jnp.transpose(p8[:, :3]).reshape(3, _NGRP, _CHUNK)
    q_r = jnp.transpose(
        jnp.pad(q2, ((0, pad_m), (0, 0)))).reshape(3, _NGRP, _CHUNK)

    pr = params
    w8 = jnp.zeros((8, 2 * _HID), jnp.float32).at[:3].set(pr["fc_pos_W"])

    def r1(x):
        return x.reshape(1, -1)

    def convk(w):
        return jnp.transpose(w, (2, 3, 1, 0)).reshape(9, _FEAT, _FEAT)

    zeros48 = jnp.zeros((_GROWS, _HID + 16), jnp.float32)
    zeros80 = jnp.zeros((_GROWS, _FEAT + 16), jnp.float32)

    net, idx3, tapidx, tapw = _run_stage_a(
        p8, p_r, q_r, w8, r1(pr["fc_pos_b"]),
        pr["blk0_W0"], r1(pr["blk0_b0"]),
        pr["blk0_W1"], r1(pr["blk0_b1"]), pr["blk0_Ws"], n)
    idx_xz = idx3[:1]

    for i in (1, 2):
        sums = _scatter1(net, idx_xz, zeros48)[0]
        pooled = _gather_mean(sums, idx_xz[0])
        with_fc = i == 2
        net = _run_block(net, pooled,
                         pr["blk%d_W0" % i], r1(pr["blk%d_b0" % i]),
                         pr["blk%d_W1" % i], r1(pr["blk%d_b1" % i]),
                         pr["blk%d_Ws" % i],
                         pr["fc_c_W"], r1(pr["fc_c_b"]), with_fc)

    s0, s1, s2 = _scatter3(net, idx3, zeros80)
    sums3 = jnp.stack([s0, s1, s2])
    tabs = _run_convs(sums3, convk(pr["conv1_W"]), r1(pr["conv1_b"]),
                      convk(pr["conv2_W"]), r1(pr["conv2_b"]))

    out_cm = _sample(tabs, tapidx, tapw)
    return jnp.transpose(out_cm[:, :m])[None]


# speedup vs baseline: 1.6333x; 1.0757x over previous
import functools

import jax
import jax.numpy as jnp
from jax import lax
from jax.experimental import pallas as pl
from jax.experimental.pallas import tpu as pltpu
from jax.experimental.pallas import tpu_sc as plsc

_FEAT = 64
_HID = 32
_RESO = 64
_PAD = 0.1
_NB = _RESO * _RESO
_GROWS = 4224
_NC, _NS = 2, 16
_NW = _NC * _NS
_CHUNK = 128
_NPAD = 102400
_ROWS_PER_W = _NPAD // _NW
_NCHUNKS = _ROWS_PER_W // _CHUNK
_RB = 2048


def _sc_mesh():
    return plsc.VectorSubcoreMesh(
        core_axis_name="c", subcore_axis_name="s",
        num_cores=_NC, num_subcores=_NS)


def _make_scatter(nplanes, width):
    mesh = _sc_mesh()
    out_type = [jax.ShapeDtypeStruct((_NC, _GROWS, width), jnp.float32)
                for _ in range(nplanes)]
    scratch = ([pltpu.VMEM((nplanes, _NCHUNKS, _CHUNK), jnp.int32)]
               + [pltpu.VMEM((2, _CHUNK, width), jnp.float32)]
               + [pltpu.VMEM_SHARED((_GROWS, width), jnp.float32)
                  for _ in range(nplanes)]
               + [pltpu.SemaphoreType.DMA, pltpu.SemaphoreType.DMA])

    @functools.partial(pl.kernel, out_type=out_type, mesh=mesh,
                       scratch_types=scratch,
                       compiler_params=pltpu.CompilerParams(
                           use_tc_tiling_on_sc=False))
    def k(feats_hbm, idx_hbm, zeros_hbm, *rest):
        outs = rest[:nplanes]
        idx_v = rest[nplanes]
        rows_v = rest[nplanes + 1]
        grids = rest[nplanes + 2:nplanes + 2 + nplanes]
        lsem, ssem = rest[nplanes + 2 + nplanes:]
        cid = lax.axis_index("c")
        sid = lax.axis_index("s")
        wid = sid * _NC + cid

        pltpu.sync_copy(idx_hbm.at[:, pl.ds(wid * _NCHUNKS, _NCHUNKS)],
                        idx_v)
        rpt = _GROWS // _NS
        for p in range(nplanes):
            pltpu.sync_copy(zeros_hbm.at[pl.ds(sid * rpt, rpt)],
                            grids[p].at[pl.ds(sid * rpt, rpt)])
        plsc.subcore_barrier()

        def fsrc(j):
            return feats_hbm.at[pl.ds(wid * _ROWS_PER_W + j * _CHUNK,
                                      _CHUNK)]

        pltpu.async_copy(fsrc(0), rows_v.at[0], lsem)

        def body(j, _):
            buf = j % 2
            pltpu.make_async_copy(fsrc(j), rows_v.at[buf], lsem).wait()
            @pl.when(j >= 1)
            def _():
                for p in range(nplanes):
                    pltpu.make_async_copy(
                        rows_v.at[1 - buf],
                        grids[p].at[idx_v.at[p, j - 1]], ssem).wait()
            @pl.when(j < _NCHUNKS - 1)
            def _():
                pltpu.async_copy(fsrc(j + 1), rows_v.at[1 - buf], lsem)
            for p in range(nplanes):
                pltpu.async_copy(rows_v.at[buf],
                                 grids[p].at[idx_v.at[p, j]], ssem,
                                 add=True)
            return 0
        lax.fori_loop(0, _NCHUNKS, body, 0)
        for p in range(nplanes):
            pltpu.make_async_copy(
                rows_v.at[(_NCHUNKS - 1) % 2],
                grids[p].at[idx_v.at[p, _NCHUNKS - 1]], ssem).wait()

        plsc.subcore_barrier()
        for p in range(nplanes):
            pltpu.sync_copy(grids[p].at[pl.ds(sid * rpt, rpt)],
                            outs[p].at[cid, pl.ds(sid * rpt, rpt)])

    return k


@functools.cache
def _get_scatter(nplanes, width):
    return _make_scatter(nplanes, width)


def _scatter1(feats, idx, zeros):
    return _get_scatter(1, _HID + 16)(feats, idx, zeros)


def _scatter3(feats, idx, zeros):
    return _get_scatter(3, _FEAT + 16)(feats, idx, zeros)


def _make_gather_mean():
    mesh = _sc_mesh()
    w = _HID + 16

    @functools.partial(
        pl.kernel, mesh=mesh,
        out_type=jax.ShapeDtypeStruct((_NPAD, _HID), jnp.float32),
        scratch_types=[pltpu.VMEM((_NCHUNKS, _CHUNK), jnp.int32),
                       pltpu.VMEM((2, _CHUNK, _HID), jnp.float32),
                       pltpu.VMEM((_NC, _GROWS // _NS, w), jnp.float32),
                       pltpu.VMEM((_GROWS // _NS, _HID), jnp.float32),
                       pltpu.VMEM_SHARED((_GROWS, _HID), jnp.float32),
                       pltpu.SemaphoreType.DMA,
                       pltpu.SemaphoreType.DMA],
        compiler_params=pltpu.CompilerParams(use_tc_tiling_on_sc=False,
                                             needs_layout_passes=False))
    def k(sums_hbm, idx_hbm, out_hbm, idx_v, o_v, part_v, mean_v, mean_sp,
          gsem, osem):
        cid = lax.axis_index("c")
        sid = lax.axis_index("s")
        wid = sid * _NC + cid
        rpt = _GROWS // _NS

        r0 = sid * rpt
        for c_ in range(_NC):
            pltpu.sync_copy(sums_hbm.at[c_, pl.ds(r0, rpt)], part_v.at[c_])

        def norm_row(r, _):
            cntv = (part_v[0, r, pl.ds(_HID, 16)]
                    + part_v[1, r, pl.ds(_HID, 16)])
            rv = 1.0 / jnp.maximum(jnp.full((16,), cntv[0], jnp.float32),
                                   1.0)
            for h in range(_HID // 16):
                a = (part_v[0, r, pl.ds(h * 16, 16)]
                     + part_v[1, r, pl.ds(h * 16, 16)])
                mean_v[r, pl.ds(h * 16, 16)] = a * rv
            return 0
        lax.fori_loop(0, rpt, norm_row, 0)
        pltpu.sync_copy(mean_v, mean_sp.at[pl.ds(r0, rpt)])
        pltpu.sync_copy(idx_hbm.at[pl.ds(wid * _NCHUNKS, _NCHUNKS)], idx_v)
        plsc.subcore_barrier()

        def fire(j, buf):
            pltpu.async_copy(mean_sp.at[idx_v.at[j]], o_v.at[buf], gsem)

        def wait_g(j, buf):
            pltpu.make_async_copy(mean_sp.at[idx_v.at[j]], o_v.at[buf],
                                  gsem).wait()

        def odst(j):
            return out_hbm.at[pl.ds(wid * _ROWS_PER_W + j * _CHUNK, _CHUNK)]

        fire(0, 0)

        def body(j, _):
            buf = j % 2
            nbuf = (j + 1) % 2
            wait_g(j, buf)
            @pl.when(j >= 1)
            def _():
                pltpu.make_async_copy(o_v.at[nbuf], odst(j - 1),
                                      osem).wait()
            @pl.when(j < _NCHUNKS - 1)
            def _():
                fire(j + 1, nbuf)
            pltpu.async_copy(o_v.at[buf], odst(j), osem)
            return 0
        lax.fori_loop(0, _NCHUNKS, body, 0)

        pltpu.make_async_copy(o_v.at[(_NCHUNKS - 1) % 2],
                              odst(_NCHUNKS - 1), osem).wait()

    return k


@functools.cache
def _get_gather_mean():
    return _make_gather_mean()


def _gather_mean(sums, idx):
    return _get_gather_mean()(sums, idx)


_NGRP = _NPAD // _CHUNK
_QPARTS, _CPARTS = 4, 8
_CPT = _FEAT // _CPARTS
_CSTRIDE = _CPT + 1
_QPT = _NPAD // _QPARTS
_GPT = _QPT // _CHUNK


def _make_sampler():
    mesh = _sc_mesh()

    @functools.partial(
        pl.kernel, mesh=mesh,
        out_type=jax.ShapeDtypeStruct((_FEAT, _NPAD), jnp.float32),
        scratch_types=[pltpu.VMEM((3, _NB * _CSTRIDE), jnp.float32),
                       pltpu.VMEM((512, _CPT), jnp.float32),
                       pltpu.VMEM((2, _CPT, _CHUNK), jnp.float32),
                       pltpu.VMEM((2, 12, _CHUNK), jnp.int32),
                       pltpu.VMEM((2, 12, _CHUNK), jnp.float32),
                       pltpu.SemaphoreType.DMA,
                       pltpu.SemaphoreType.DMA],
        compiler_params=pltpu.CompilerParams(use_tc_tiling_on_sc=False,
                                             needs_layout_passes=False))
    def k(tabs_hbm, idx_hbm, w_hbm, out_hbm,
          tab_v, stage_v, out_g, idx_v, w_v, sem, osem):
        cid = lax.axis_index("c")
        sid = lax.axis_index("s")
        wid = sid * _NC + cid
        qpart = wid // _CPARTS
        cpart = wid % _CPARTS
        c0 = cpart * _CPT
        g0 = qpart * _GPT

        for p in range(3):
            def stage_piece(s, _):
                pltpu.sync_copy(
                    tabs_hbm.at[p, pl.ds(s * 512, 512), pl.ds(c0, _CPT)],
                    stage_v)

                def spread(r, _):
                    rows = lax.iota(jnp.int32, 16) + r * 16
                    for c in range(_CPT):
                        cs = jnp.full((16,), c, jnp.int32)
                        v = plsc.load_gather(stage_v, [rows, cs])
                        plsc.store_scatter(
                            tab_v.at[p],
                            [(rows + s * 512) * _CSTRIDE + c], v)
                    return 0
                lax.fori_loop(0, 32, spread, 0)
                return 0
            lax.fori_loop(0, _NB // 512, stage_piece, 0)

        def issue(g, buf):
            pltpu.async_copy(idx_hbm.at[g0 + g], idx_v.at[buf], sem)
            pltpu.async_copy(w_hbm.at[g0 + g], w_v.at[buf], sem)

        def wait_in(g, buf):
            pltpu.make_async_copy(idx_hbm.at[g0 + g], idx_v.at[buf],
                                  sem).wait()
            pltpu.make_async_copy(w_hbm.at[g0 + g], w_v.at[buf],
                                  sem).wait()

        def out_dst(g, buf):
            qstart = qpart * _QPT + g * _CHUNK
            return out_hbm.at[pl.ds(c0, _CPT), pl.ds(qstart, _CHUNK)]

        issue(0, 0)

        def group_body(g, _):
            buf = g % 2
            @pl.when(g < _GPT - 1)
            def _():
                issue(g + 1, (g + 1) % 2)
            wait_in(g, buf)
            @pl.when(g >= 2)
            def _():
                pltpu.make_async_copy(out_g.at[buf], out_dst(g - 2, buf),
                                      osem).wait()
            cur_i = idx_v.at[buf]
            cur_w = w_v.at[buf]
            for qb in range(_CHUNK // 16):
                qi = [cur_i[t, pl.ds(qb * 16, 16)] * _CSTRIDE
                      for t in range(12)]
                wv = [cur_w[t, pl.ds(qb * 16, 16)] for t in range(12)]
                for cb in range(_CPT // 4):
                    cs = [cb * 4 + i for i in range(4)]
                    accs = [wv[0] * plsc.load_gather(tab_v.at[0],
                                                     [qi[0] + c])
                            for c in cs]
                    for t in range(1, 12):
                        gs = [plsc.load_gather(tab_v.at[t // 4],
                                               [qi[t] + c]) for c in cs]
                        accs = [a + wv[t] * g for a, g in zip(accs, gs)]
                    for c, a in zip(cs, accs):
                        out_g[buf, c, pl.ds(qb * 16, 16)] = a
            pltpu.async_copy(out_g.at[buf], out_dst(g, buf), osem)
            return 0
        lax.fori_loop(0, _GPT, group_body, 0)

        pltpu.make_async_copy(out_g.at[_GPT % 2],
                              out_dst(_GPT - 2, _GPT % 2), osem).wait()
        pltpu.make_async_copy(out_g.at[(_GPT - 1) % 2],
                              out_dst(_GPT - 1, (_GPT - 1) % 2),
                              osem).wait()

    return k


@functools.cache
def _get_sampler():
    return _make_sampler()


def _sample(tabs, tapidx, tapw):
    return _get_sampler()(tabs, tapidx, tapw)


_GRB = _RB // _CHUNK

_PLANE_DIMS = ((0, 2), (0, 1), (1, 2))


def _stage_a_body(p_ref, pr_ref, qr_ref, w8_ref, b_ref, w0_ref, b0_ref,
                  w1_ref, b1_ref, ws_ref, o_ref, oi_ref, oti_ref, otw_ref,
                  *, n):
    x = jnp.dot(p_ref[...], w8_ref[...], preferred_element_type=jnp.float32)
    x = x + b_ref[...]
    h = jax.nn.relu(jnp.dot(x, w0_ref[...], preferred_element_type=jnp.float32)
                    + b0_ref[...])
    dx = jax.nn.relu(jnp.dot(h, w1_ref[...], preferred_element_type=jnp.float32)
                     + b1_ref[...])
    net = jnp.dot(x, ws_ref[...], preferred_element_type=jnp.float32) + dx
    ones = jnp.ones((_RB, 1), jnp.float32)
    zeros = jnp.zeros((_RB, 15), jnp.float32)
    o_ref[...] = jnp.concatenate([net, ones, zeros], axis=1)

    base = pl.program_id(0) * _RB
    gr = (base + lax.broadcasted_iota(jnp.int32, (_GRB, _CHUNK), 0) * _CHUNK
          + lax.broadcasted_iota(jnp.int32, (_GRB, _CHUNK), 1))
    pc = [jnp.clip((pr_ref[d] + 0.5) / (1.0 + _PAD + 1e-6),
                   0.0, 1.0 - 1e-6) for d in range(3)]
    for d_i, d in enumerate(_PLANE_DIMS):
        xi0 = jnp.floor(pc[d[0]] * _RESO).astype(jnp.int32)
        xi1 = jnp.floor(pc[d[1]] * _RESO).astype(jnp.int32)
        oi_ref[d_i] = jnp.where(gr < n, xi0 + _RESO * xi1, _NB)

    for d_i, d in enumerate(_PLANE_DIMS):
        xq = qr_ref[d[0]] * _RESO - 0.5
        yq = qr_ref[d[1]] * _RESO - 0.5
        x0 = jnp.floor(xq)
        y0 = jnp.floor(yq)
        wx = xq - x0
        wy = yq - y0
        x0i = jnp.clip(x0, 0, _RESO - 1).astype(jnp.int32)
        x1i = jnp.clip(x0 + 1.0, 0, _RESO - 1).astype(jnp.int32)
        y0i = jnp.clip(y0, 0, _RESO - 1).astype(jnp.int32)
        y1i = jnp.clip(y0 + 1.0, 0, _RESO - 1).astype(jnp.int32)
        taps = ((y0i, x0i, (1 - wx) * (1 - wy)),
                (y0i, x1i, wx * (1 - wy)),
                (y1i, x0i, (1 - wx) * wy),
                (y1i, x1i, wx * wy))
        for t, (yi, xi, wt) in enumerate(taps):
            oti_ref[:, 4 * d_i + t] = yi * _RESO + xi
            otw_ref[:, 4 * d_i + t] = wt


def _run_stage_a(p8, p_r, q_r, w8, b, w0, b0, w1, b1, ws, n):
    grid = _NPAD // _RB
    full = lambda shp: pl.BlockSpec(shp, lambda i: (0, 0))
    return pl.pallas_call(
        functools.partial(_stage_a_body, n=n),
        grid=(grid,),
        in_specs=[pl.BlockSpec((_RB, 8), lambda i: (i, 0)),
                  pl.BlockSpec((3, _GRB, _CHUNK), lambda i: (0, i, 0)),
                  pl.BlockSpec((3, _GRB, _CHUNK), lambda i: (0, i, 0)),
                  full((8, 2 * _HID)), full((1, 2 * _HID)),
                  full((2 * _HID, _HID)), full((1, _HID)),
                  full((_HID, _HID)), full((1, _HID)),
                  full((2 * _HID, _HID))],
        out_specs=[pl.BlockSpec((_RB, _HID + 16), lambda i: (i, 0)),
                   pl.BlockSpec((3, _GRB, _CHUNK), lambda i: (0, i, 0)),
                   pl.BlockSpec((_GRB, 12, _CHUNK), lambda i: (i, 0, 0)),
                   pl.BlockSpec((_GRB, 12, _CHUNK), lambda i: (i, 0, 0))],
        out_shape=[jax.ShapeDtypeStruct((_NPAD, _HID + 16), jnp.float32),
                   jax.ShapeDtypeStruct((3, _NGRP, _CHUNK), jnp.int32),
                   jax.ShapeDtypeStruct((_NGRP, 12, _CHUNK), jnp.int32),
                   jax.ShapeDtypeStruct((_NGRP, 12, _CHUNK), jnp.float32)],
    )(p8, p_r, q_r, w8, b, w0, b0, w1, b1, ws)


def _block_body(net_ref, pool_ref, w0_ref, b0_ref, w1_ref, b1_ref, ws_ref,
                fcw_ref, fcb_ref, o_ref, *, out_width, with_fc):
    net = net_ref[:, :_HID]
    pool = pool_ref[...]
    w0 = w0_ref[...]
    ws = ws_ref[...]
    xw0 = (jnp.dot(net, w0[:_HID], preferred_element_type=jnp.float32)
           + jnp.dot(pool, w0[_HID:], preferred_element_type=jnp.float32))
    h = jax.nn.relu(xw0 + b0_ref[...])
    dx = jax.nn.relu(jnp.dot(h, w1_ref[...], preferred_element_type=jnp.float32)
                     + b1_ref[...])
    out = (jnp.dot(net, ws[:_HID], preferred_element_type=jnp.float32)
           + jnp.dot(pool, ws[_HID:], preferred_element_type=jnp.float32)
           + dx)
    if with_fc:
        out = jnp.dot(out, fcw_ref[...], preferred_element_type=jnp.float32) \
            + fcb_ref[...]
        width = _FEAT
    else:
        width = _HID
    ones = jnp.ones((_RB, 1), jnp.float32)
    zeros = jnp.zeros((_RB, out_width - width - 1), jnp.float32)
    o_ref[...] = jnp.concatenate([out, ones, zeros], axis=1)


def _run_block(net, pool, w0, b0, w1, b1, ws, fcw, fcb, with_fc):
    out_width = (_FEAT + 16) if with_fc else (_HID + 16)
    grid = _NPAD // _RB
    full = lambda shp: pl.BlockSpec(shp, lambda i: (0, 0))
    body = functools.partial(_block_body, out_width=out_width, with_fc=with_fc)
    return pl.pallas_call(
        body,
        grid=(grid,),
        in_specs=[pl.BlockSpec((_RB, _HID + 16), lambda i: (i, 0)),
                  pl.BlockSpec((_RB, _HID), lambda i: (i, 0)),
                  full((2 * _HID, _HID)), full((1, _HID)),
                  full((_HID, _HID)), full((1, _HID)),
                  full((2 * _HID, _HID)),
                  full((_HID, _FEAT)), full((1, _FEAT))],
        out_specs=pl.BlockSpec((_RB, out_width), lambda i: (i, 0)),
        out_shape=jax.ShapeDtypeStruct((_NPAD, out_width), jnp.float32),
    )(net, pool, w0, b0, w1, b1, ws, fcw, fcb)


def _conv_taps(x, wk, bias, col):
    acc = jnp.zeros((_NB, _FEAT), jnp.float32)
    xp = jnp.concatenate([jnp.zeros((_RESO + 1, _FEAT), jnp.float32), x,
                          jnp.zeros((_RESO + 1, _FEAT), jnp.float32)], axis=0)
    k = 0
    for di in (-1, 0, 1):
        for dj in (-1, 0, 1):
            s = di * _RESO + dj
            tap = lax.slice_in_dim(xp, _RESO + 1 + s, _RESO + 1 + s + _NB,
                                   axis=0)
            if dj == -1:
                tap = jnp.where(col >= 1, tap, 0.0)
            elif dj == 1:
                tap = jnp.where(col <= _RESO - 2, tap, 0.0)
            acc = acc + jnp.dot(tap, wk[k],
                                preferred_element_type=jnp.float32)
            k += 1
    return jax.nn.relu(acc + bias)


def _conv_body(s_ref, w1_ref, b1_ref, w2_ref, b2_ref, o_ref):
    s = s_ref[0, 0] + s_ref[0, 1]
    s = s[:_NB]
    mean = s[:, :_FEAT] / jnp.maximum(s[:, _FEAT:_FEAT + 1], 1.0)
    col = lax.broadcasted_iota(jnp.int32, (_NB, _FEAT), 0) % _RESO
    h = _conv_taps(mean, w1_ref, b1_ref[...], col)
    o_ref[0] = _conv_taps(h, w2_ref, b2_ref[...], col)


def _run_convs(sums3, w1k, b1, w2k, b2):
    full = lambda shp: pl.BlockSpec(shp, lambda i: tuple(0 for _ in shp))
    return pl.pallas_call(
        _conv_body,
        grid=(3,),
        in_specs=[pl.BlockSpec((1, _NC, _GROWS, _FEAT + 16),
                               lambda i: (i, 0, 0, 0)),
                  full((9, _FEAT, _FEAT)), full((1, _FEAT)),
                  full((9, _FEAT, _FEAT)), full((1, _FEAT))],
        out_specs=pl.BlockSpec((1, _NB, _FEAT), lambda i: (i, 0, 0)),
        out_shape=jax.ShapeDtypeStruct((3, _NB, _FEAT), jnp.float32),
    )(sums3, w1k, b1, w2k, b2)


def kernel(p, query, params):
    n = p.shape[1]
    m = query.shape[1]
    p2 = p[0]
    q2 = query[0]

    pad_n = _NPAD - n
    pad_m = _NPAD - m
    p8 = jnp.pad(p2, ((0, pad_n), (0, 5)))
    p_r = jnp.transpose(p8[:, :3]).reshape(3, _NGRP, _CHUNK)
    q_r = jnp.transpose(
        jnp.pad(q2, ((0, pad_m), (0, 0)))).reshape(3, _NGRP, _CHUNK)

    pr = params
    w8 = jnp.zeros((8, 2 * _HID), jnp.float32).at[:3].set(pr["fc_pos_W"])

    def r1(x):
        return x.reshape(1, -1)

    def convk(w):
        return jnp.transpose(w, (2, 3, 1, 0)).reshape(9, _FEAT, _FEAT)

    zeros48 = jnp.zeros((_GROWS, _HID + 16), jnp.float32)
    zeros80 = jnp.zeros((_GROWS, _FEAT + 16), jnp.float32)

    net, idx3, tapidx, tapw = _run_stage_a(
        p8, p_r, q_r, w8, r1(pr["fc_pos_b"]),
        pr["blk0_W0"], r1(pr["blk0_b0"]),
        pr["blk0_W1"], r1(pr["blk0_b1"]), pr["blk0_Ws"], n)
    idx_xz = idx3[:1]

    for i in (1, 2):
        sums = _scatter1(net, idx_xz, zeros48)[0]
        pooled = _gather_mean(sums, idx_xz[0])
        with_fc = i == 2
        net = _run_block(net, pooled,
                         pr["blk%d_W0" % i], r1(pr["blk%d_b0" % i]),
                         pr["blk%d_W1" % i], r1(pr["blk%d_b1" % i]),
                         pr["blk%d_Ws" % i],
                         pr["fc_c_W"], r1(pr["fc_c_b"]), with_fc)

    s0, s1, s2 = _scatter3(net, idx3, zeros80)
    sums3 = jnp.stack([s0, s1, s2])
    tabs = _run_convs(sums3, convk(pr["conv1_W"]), r1(pr["conv1_b"]),
                      convk(pr["conv2_W"]), r1(pr["conv2_b"]))

    out_cm = _sample(tabs, tapidx, tapw)
    return jnp.transpose(out_cm[:, :m])[None]


# speedup vs baseline: 1.7193x; 1.0526x over previous
import functools

import jax
import jax.numpy as jnp
from jax import lax
from jax.experimental import pallas as pl
from jax.experimental.pallas import tpu as pltpu
from jax.experimental.pallas import tpu_sc as plsc

_FEAT = 64
_HID = 32
_RESO = 64
_PAD = 0.1
_NB = _RESO * _RESO
_GROWS = 4224
_NC, _NS = 2, 16
_NW = _NC * _NS
_CHUNK = 128
_NPAD = 102400
_ROWS_PER_W = _NPAD // _NW
_NCHUNKS = _ROWS_PER_W // _CHUNK
_RB = 4096


def _sc_mesh():
    return plsc.VectorSubcoreMesh(
        core_axis_name="c", subcore_axis_name="s",
        num_cores=_NC, num_subcores=_NS)


def _make_scatter(nplanes, width):
    mesh = _sc_mesh()
    out_type = [jax.ShapeDtypeStruct((_NC, _GROWS, width), jnp.float32)
                for _ in range(nplanes)]
    scratch = ([pltpu.VMEM((nplanes, _NCHUNKS, _CHUNK), jnp.int32)]
               + [pltpu.VMEM((2, _CHUNK, width), jnp.float32)]
               + [pltpu.VMEM_SHARED((_GROWS, width), jnp.float32)
                  for _ in range(nplanes)]
               + [pltpu.SemaphoreType.DMA, pltpu.SemaphoreType.DMA])

    @functools.partial(pl.kernel, out_type=out_type, mesh=mesh,
                       scratch_types=scratch,
                       compiler_params=pltpu.CompilerParams(
                           use_tc_tiling_on_sc=False))
    def k(feats_hbm, idx_hbm, zeros_hbm, *rest):
        outs = rest[:nplanes]
        idx_v = rest[nplanes]
        rows_v = rest[nplanes + 1]
        grids = rest[nplanes + 2:nplanes + 2 + nplanes]
        lsem, ssem = rest[nplanes + 2 + nplanes:]
        cid = lax.axis_index("c")
        sid = lax.axis_index("s")
        wid = sid * _NC + cid

        pltpu.sync_copy(idx_hbm.at[:, pl.ds(wid * _NCHUNKS, _NCHUNKS)],
                        idx_v)
        rpt = _GROWS // _NS
        for p in range(nplanes):
            pltpu.sync_copy(zeros_hbm.at[pl.ds(sid * rpt, rpt)],
                            grids[p].at[pl.ds(sid * rpt, rpt)])
        plsc.subcore_barrier()

        def fsrc(j):
            return feats_hbm.at[pl.ds(wid * _ROWS_PER_W + j * _CHUNK,
                                      _CHUNK)]

        pltpu.async_copy(fsrc(0), rows_v.at[0], lsem)

        def body(j, _):
            buf = j % 2
            pltpu.make_async_copy(fsrc(j), rows_v.at[buf], lsem).wait()
            @pl.when(j >= 1)
            def _():
                for p in range(nplanes):
                    pltpu.make_async_copy(
                        rows_v.at[1 - buf],
                        grids[p].at[idx_v.at[p, j - 1]], ssem).wait()
            @pl.when(j < _NCHUNKS - 1)
            def _():
                pltpu.async_copy(fsrc(j + 1), rows_v.at[1 - buf], lsem)
            for p in range(nplanes):
                pltpu.async_copy(rows_v.at[buf],
                                 grids[p].at[idx_v.at[p, j]], ssem,
                                 add=True)
            return 0
        lax.fori_loop(0, _NCHUNKS, body, 0)
        for p in range(nplanes):
            pltpu.make_async_copy(
                rows_v.at[(_NCHUNKS - 1) % 2],
                grids[p].at[idx_v.at[p, _NCHUNKS - 1]], ssem).wait()

        plsc.subcore_barrier()
        for p in range(nplanes):
            pltpu.sync_copy(grids[p].at[pl.ds(sid * rpt, rpt)],
                            outs[p].at[cid, pl.ds(sid * rpt, rpt)])

    return k


@functools.cache
def _get_scatter(nplanes, width):
    return _make_scatter(nplanes, width)


def _scatter1(feats, idx, zeros):
    return _get_scatter(1, _HID + 16)(feats, idx, zeros)


def _scatter3(feats, idx, zeros):
    return _get_scatter(3, _FEAT + 16)(feats, idx, zeros)


def _make_gather_mean():
    mesh = _sc_mesh()
    w = _HID + 16

    @functools.partial(
        pl.kernel, mesh=mesh,
        out_type=jax.ShapeDtypeStruct((_NPAD, _HID), jnp.float32),
        scratch_types=[pltpu.VMEM((_NCHUNKS, _CHUNK), jnp.int32),
                       pltpu.VMEM((2, _CHUNK, _HID), jnp.float32),
                       pltpu.VMEM((_NC, _GROWS // _NS, w), jnp.float32),
                       pltpu.VMEM((_GROWS // _NS, _HID), jnp.float32),
                       pltpu.VMEM_SHARED((_GROWS, _HID), jnp.float32),
                       pltpu.SemaphoreType.DMA,
                       pltpu.SemaphoreType.DMA],
        compiler_params=pltpu.CompilerParams(use_tc_tiling_on_sc=False,
                                             needs_layout_passes=False))
    def k(sums_hbm, idx_hbm, out_hbm, idx_v, o_v, part_v, mean_v, mean_sp,
          gsem, osem):
        cid = lax.axis_index("c")
        sid = lax.axis_index("s")
        wid = sid * _NC + cid
        rpt = _GROWS // _NS

        r0 = sid * rpt
        for c_ in range(_NC):
            pltpu.sync_copy(sums_hbm.at[c_, pl.ds(r0, rpt)], part_v.at[c_])

        def norm_row(r, _):
            cntv = (part_v[0, r, pl.ds(_HID, 16)]
                    + part_v[1, r, pl.ds(_HID, 16)])
            rv = 1.0 / jnp.maximum(jnp.full((16,), cntv[0], jnp.float32),
                                   1.0)
            for h in range(_HID // 16):
                a = (part_v[0, r, pl.ds(h * 16, 16)]
                     + part_v[1, r, pl.ds(h * 16, 16)])
                mean_v[r, pl.ds(h * 16, 16)] = a * rv
            return 0
        lax.fori_loop(0, rpt, norm_row, 0)
        pltpu.sync_copy(mean_v, mean_sp.at[pl.ds(r0, rpt)])
        pltpu.sync_copy(idx_hbm.at[pl.ds(wid * _NCHUNKS, _NCHUNKS)], idx_v)
        plsc.subcore_barrier()

        def fire(j, buf):
            pltpu.async_copy(mean_sp.at[idx_v.at[j]], o_v.at[buf], gsem)

        def wait_g(j, buf):
            pltpu.make_async_copy(mean_sp.at[idx_v.at[j]], o_v.at[buf],
                                  gsem).wait()

        def odst(j):
            return out_hbm.at[pl.ds(wid * _ROWS_PER_W + j * _CHUNK, _CHUNK)]

        fire(0, 0)

        def body(j, _):
            buf = j % 2
            nbuf = (j + 1) % 2
            wait_g(j, buf)
            @pl.when(j >= 1)
            def _():
                pltpu.make_async_copy(o_v.at[nbuf], odst(j - 1),
                                      osem).wait()
            @pl.when(j < _NCHUNKS - 1)
            def _():
                fire(j + 1, nbuf)
            pltpu.async_copy(o_v.at[buf], odst(j), osem)
            return 0
        lax.fori_loop(0, _NCHUNKS, body, 0)

        pltpu.make_async_copy(o_v.at[(_NCHUNKS - 1) % 2],
                              odst(_NCHUNKS - 1), osem).wait()

    return k


@functools.cache
def _get_gather_mean():
    return _make_gather_mean()


def _gather_mean(sums, idx):
    return _get_gather_mean()(sums, idx)


_NGRP = _NPAD // _CHUNK
_QPARTS, _CPARTS = 4, 8
_CPT = _FEAT // _CPARTS
_CSTRIDE = _CPT + 1
_QPT = _NPAD // _QPARTS
_GPT = _QPT // _CHUNK


def _make_sampler():
    mesh = _sc_mesh()

    @functools.partial(
        pl.kernel, mesh=mesh,
        out_type=jax.ShapeDtypeStruct((_FEAT, _NPAD), jnp.float32),
        scratch_types=[pltpu.VMEM((3, _NB * _CSTRIDE), jnp.float32),
                       pltpu.VMEM((512, _CPT), jnp.float32),
                       pltpu.VMEM((2, _CPT, _CHUNK), jnp.float32),
                       pltpu.VMEM((2, 12, _CHUNK), jnp.int32),
                       pltpu.VMEM((2, 12, _CHUNK), jnp.float32),
                       pltpu.SemaphoreType.DMA,
                       pltpu.SemaphoreType.DMA],
        compiler_params=pltpu.CompilerParams(use_tc_tiling_on_sc=False,
                                             needs_layout_passes=False))
    def k(tabs_hbm, idx_hbm, w_hbm, out_hbm,
          tab_v, stage_v, out_g, idx_v, w_v, sem, osem):
        cid = lax.axis_index("c")
        sid = lax.axis_index("s")
        wid = sid * _NC + cid
        qpart = wid // _CPARTS
        cpart = wid % _CPARTS
        c0 = cpart * _CPT
        g0 = qpart * _GPT

        for p in range(3):
            def stage_piece(s, _):
                pltpu.sync_copy(
                    tabs_hbm.at[p, pl.ds(s * 512, 512), pl.ds(c0, _CPT)],
                    stage_v)

                def spread(r, _):
                    rows = lax.iota(jnp.int32, 16) + r * 16
                    for c in range(_CPT):
                        cs = jnp.full((16,), c, jnp.int32)
                        v = plsc.load_gather(stage_v, [rows, cs])
                        plsc.store_scatter(
                            tab_v.at[p],
                            [(rows + s * 512) * _CSTRIDE + c], v)
                    return 0
                lax.fori_loop(0, 32, spread, 0)
                return 0
            lax.fori_loop(0, _NB // 512, stage_piece, 0)

        def issue(g, buf):
            pltpu.async_copy(idx_hbm.at[g0 + g], idx_v.at[buf], sem)
            pltpu.async_copy(w_hbm.at[g0 + g], w_v.at[buf], sem)

        def wait_in(g, buf):
            pltpu.make_async_copy(idx_hbm.at[g0 + g], idx_v.at[buf],
                                  sem).wait()
            pltpu.make_async_copy(w_hbm.at[g0 + g], w_v.at[buf],
                                  sem).wait()

        def out_dst(g, buf):
            qstart = qpart * _QPT + g * _CHUNK
            return out_hbm.at[pl.ds(c0, _CPT), pl.ds(qstart, _CHUNK)]

        issue(0, 0)

        def group_body(g, _):
            buf = g % 2
            @pl.when(g < _GPT - 1)
            def _():
                issue(g + 1, (g + 1) % 2)
            wait_in(g, buf)
            @pl.when(g >= 2)
            def _():
                pltpu.make_async_copy(out_g.at[buf], out_dst(g - 2, buf),
                                      osem).wait()
            cur_i = idx_v.at[buf]
            cur_w = w_v.at[buf]
            for qb in range(_CHUNK // 16):
                qi = [cur_i[t, pl.ds(qb * 16, 16)] * _CSTRIDE
                      for t in range(12)]
                wv = [cur_w[t, pl.ds(qb * 16, 16)] for t in range(12)]
                for cb in range(_CPT // 8):
                    cs = [cb * 8 + i for i in range(8)]
                    accs = [wv[0] * plsc.load_gather(tab_v.at[0],
                                                     [qi[0] + c])
                            for c in cs]
                    for t in range(1, 12):
                        gs = [plsc.load_gather(tab_v.at[t // 4],
                                               [qi[t] + c]) for c in cs]
                        accs = [a + wv[t] * g for a, g in zip(accs, gs)]
                    for c, a in zip(cs, accs):
                        out_g[buf, c, pl.ds(qb * 16, 16)] = a
            pltpu.async_copy(out_g.at[buf], out_dst(g, buf), osem)
            return 0
        lax.fori_loop(0, _GPT, group_body, 0)

        pltpu.make_async_copy(out_g.at[_GPT % 2],
                              out_dst(_GPT - 2, _GPT % 2), osem).wait()
        pltpu.make_async_copy(out_g.at[(_GPT - 1) % 2],
                              out_dst(_GPT - 1, (_GPT - 1) % 2),
                              osem).wait()

    return k


@functools.cache
def _get_sampler():
    return _make_sampler()


def _sample(tabs, tapidx, tapw):
    return _get_sampler()(tabs, tapidx, tapw)


_GRB = _RB // _CHUNK

_PLANE_DIMS = ((0, 2), (0, 1), (1, 2))


def _stage_a_body(p_ref, pr_ref, qr_ref, w8_ref, b_ref, w0_ref, b0_ref,
                  w1_ref, b1_ref, ws_ref, o_ref, oi_ref, oti_ref, otw_ref,
                  *, n):
    x = jnp.dot(p_ref[...], w8_ref[...], preferred_element_type=jnp.float32)
    x = x + b_ref[...]
    h = jax.nn.relu(jnp.dot(x, w0_ref[...], preferred_element_type=jnp.float32)
                    + b0_ref[...])
    dx = jax.nn.relu(jnp.dot(h, w1_ref[...], preferred_element_type=jnp.float32)
                     + b1_ref[...])
    net = jnp.dot(x, ws_ref[...], preferred_element_type=jnp.float32) + dx
    ones = jnp.ones((_RB, 1), jnp.float32)
    zeros = jnp.zeros((_RB, 15), jnp.float32)
    o_ref[...] = jnp.concatenate([net, ones, zeros], axis=1)

    base = pl.program_id(0) * _RB
    gr = (base + lax.broadcasted_iota(jnp.int32, (_GRB, _CHUNK), 0) * _CHUNK
          + lax.broadcasted_iota(jnp.int32, (_GRB, _CHUNK), 1))
    pc = [jnp.clip((pr_ref[d] + 0.5) / (1.0 + _PAD + 1e-6),
                   0.0, 1.0 - 1e-6) for d in range(3)]
    for d_i, d in enumerate(_PLANE_DIMS):
        xi0 = jnp.floor(pc[d[0]] * _RESO).astype(jnp.int32)
        xi1 = jnp.floor(pc[d[1]] * _RESO).astype(jnp.int32)
        oi_ref[d_i] = jnp.where(gr < n, xi0 + _RESO * xi1, _NB)

    for d_i, d in enumerate(_PLANE_DIMS):
        xq = qr_ref[d[0]] * _RESO - 0.5
        yq = qr_ref[d[1]] * _RESO - 0.5
        x0 = jnp.floor(xq)
        y0 = jnp.floor(yq)
        wx = xq - x0
        wy = yq - y0
        x0i = jnp.clip(x0, 0, _RESO - 1).astype(jnp.int32)
        x1i = jnp.clip(x0 + 1.0, 0, _RESO - 1).astype(jnp.int32)
        y0i = jnp.clip(y0, 0, _RESO - 1).astype(jnp.int32)
        y1i = jnp.clip(y0 + 1.0, 0, _RESO - 1).astype(jnp.int32)
        taps = ((y0i, x0i, (1 - wx) * (1 - wy)),
                (y0i, x1i, wx * (1 - wy)),
                (y1i, x0i, (1 - wx) * wy),
                (y1i, x1i, wx * wy))
        for t, (yi, xi, wt) in enumerate(taps):
            oti_ref[:, 4 * d_i + t] = yi * _RESO + xi
            otw_ref[:, 4 * d_i + t] = wt


def _run_stage_a(p8, p_r, q_r, w8, b, w0, b0, w1, b1, ws, n):
    grid = _NPAD // _RB
    full = lambda shp: pl.BlockSpec(shp, lambda i: (0, 0))
    return pl.pallas_call(
        functools.partial(_stage_a_body, n=n),
        grid=(grid,),
        in_specs=[pl.BlockSpec((_RB, 8), lambda i: (i, 0)),
                  pl.BlockSpec((3, _GRB, _CHUNK), lambda i: (0, i, 0)),
                  pl.BlockSpec((3, _GRB, _CHUNK), lambda i: (0, i, 0)),
                  full((8, 2 * _HID)), full((1, 2 * _HID)),
                  full((2 * _HID, _HID)), full((1, _HID)),
                  full((_HID, _HID)), full((1, _HID)),
                  full((2 * _HID, _HID))],
        out_specs=[pl.BlockSpec((_RB, _HID + 16), lambda i: (i, 0)),
                   pl.BlockSpec((3, _GRB, _CHUNK), lambda i: (0, i, 0)),
                   pl.BlockSpec((_GRB, 12, _CHUNK), lambda i: (i, 0, 0)),
                   pl.BlockSpec((_GRB, 12, _CHUNK), lambda i: (i, 0, 0))],
        out_shape=[jax.ShapeDtypeStruct((_NPAD, _HID + 16), jnp.float32),
                   jax.ShapeDtypeStruct((3, _NGRP, _CHUNK), jnp.int32),
                   jax.ShapeDtypeStruct((_NGRP, 12, _CHUNK), jnp.int32),
                   jax.ShapeDtypeStruct((_NGRP, 12, _CHUNK), jnp.float32)],
    )(p8, p_r, q_r, w8, b, w0, b0, w1, b1, ws)


def _block_body(net_ref, pool_ref, w0_ref, b0_ref, w1_ref, b1_ref, ws_ref,
                fcw_ref, fcb_ref, o_ref, *, out_width, with_fc):
    net = net_ref[:, :_HID]
    pool = pool_ref[...]
    w0 = w0_ref[...]
    ws = ws_ref[...]
    xw0 = (jnp.dot(net, w0[:_HID], preferred_element_type=jnp.float32)
           + jnp.dot(pool, w0[_HID:], preferred_element_type=jnp.float32))
    h = jax.nn.relu(xw0 + b0_ref[...])
    dx = jax.nn.relu(jnp.dot(h, w1_ref[...], preferred_element_type=jnp.float32)
                     + b1_ref[...])
    out = (jnp.dot(net, ws[:_HID], preferred_element_type=jnp.float32)
           + jnp.dot(pool, ws[_HID:], preferred_element_type=jnp.float32)
           + dx)
    if with_fc:
        out = jnp.dot(out, fcw_ref[...], preferred_element_type=jnp.float32) \
            + fcb_ref[...]
        width = _FEAT
    else:
        width = _HID
    ones = jnp.ones((_RB, 1), jnp.float32)
    zeros = jnp.zeros((_RB, out_width - width - 1), jnp.float32)
    o_ref[...] = jnp.concatenate([out, ones, zeros], axis=1)


def _run_block(net, pool, w0, b0, w1, b1, ws, fcw, fcb, with_fc):
    out_width = (_FEAT + 16) if with_fc else (_HID + 16)
    grid = _NPAD // _RB
    full = lambda shp: pl.BlockSpec(shp, lambda i: (0, 0))
    body = functools.partial(_block_body, out_width=out_width, with_fc=with_fc)
    return pl.pallas_call(
        body,
        grid=(grid,),
        in_specs=[pl.BlockSpec((_RB, _HID + 16), lambda i: (i, 0)),
                  pl.BlockSpec((_RB, _HID), lambda i: (i, 0)),
                  full((2 * _HID, _HID)), full((1, _HID)),
                  full((_HID, _HID)), full((1, _HID)),
                  full((2 * _HID, _HID)),
                  full((_HID, _FEAT)), full((1, _FEAT))],
        out_specs=pl.BlockSpec((_RB, out_width), lambda i: (i, 0)),
        out_shape=jax.ShapeDtypeStruct((_NPAD, out_width), jnp.float32),
    )(net, pool, w0, b0, w1, b1, ws, fcw, fcb)


def _conv_taps(x, wk, bias, col):
    acc = jnp.zeros((_NB, _FEAT), jnp.float32)
    xp = jnp.concatenate([jnp.zeros((_RESO + 1, _FEAT), jnp.float32), x,
                          jnp.zeros((_RESO + 1, _FEAT), jnp.float32)], axis=0)
    k = 0
    for di in (-1, 0, 1):
        for dj in (-1, 0, 1):
            s = di * _RESO + dj
            tap = lax.slice_in_dim(xp, _RESO + 1 + s, _RESO + 1 + s + _NB,
                                   axis=0)
            if dj == -1:
                tap = jnp.where(col >= 1, tap, 0.0)
            elif dj == 1:
                tap = jnp.where(col <= _RESO - 2, tap, 0.0)
            acc = acc + jnp.dot(tap, wk[k],
                                preferred_element_type=jnp.float32)
            k += 1
    return jax.nn.relu(acc + bias)


def _conv_body(s_ref, w1_ref, b1_ref, w2_ref, b2_ref, o_ref):
    s = s_ref[0, 0] + s_ref[0, 1]
    s = s[:_NB]
    mean = s[:, :_FEAT] / jnp.maximum(s[:, _FEAT:_FEAT + 1], 1.0)
    col = lax.broadcasted_iota(jnp.int32, (_NB, _FEAT), 0) % _RESO
    h = _conv_taps(mean, w1_ref, b1_ref[...], col)
    o_ref[0] = _conv_taps(h, w2_ref, b2_ref[...], col)


def _run_convs(sums3, w1k, b1, w2k, b2):
    full = lambda shp: pl.BlockSpec(shp, lambda i: tuple(0 for _ in shp))
    return pl.pallas_call(
        _conv_body,
        grid=(3,),
        in_specs=[pl.BlockSpec((1, _NC, _GROWS, _FEAT + 16),
                               lambda i: (i, 0, 0, 0)),
                  full((9, _FEAT, _FEAT)), full((1, _FEAT)),
                  full((9, _FEAT, _FEAT)), full((1, _FEAT))],
        out_specs=pl.BlockSpec((1, _NB, _FEAT), lambda i: (i, 0, 0)),
        out_shape=jax.ShapeDtypeStruct((3, _NB, _FEAT), jnp.float32),
    )(sums3, w1k, b1, w2k, b2)


def kernel(p, query, params):
    n = p.shape[1]
    m = query.shape[1]
    p2 = p[0]
    q2 = query[0]

    pad_n = _NPAD - n
    pad_m = _NPAD - m
    p8 = jnp.pad(p2, ((0, pad_n), (0, 5)))
    p_r = jnp.transpose(p8[:, :3]).reshape(3, _NGRP, _CHUNK)
    q_r = jnp.transpose(
        jnp.pad(q2, ((0, pad_m), (0, 0)))).reshape(3, _NGRP, _CHUNK)

    pr = params
    w8 = jnp.zeros((8, 2 * _HID), jnp.float32).at[:3].set(pr["fc_pos_W"])

    def r1(x):
        return x.reshape(1, -1)

    def convk(w):
        return jnp.transpose(w, (2, 3, 1, 0)).reshape(9, _FEAT, _FEAT)

    zeros48 = jnp.zeros((_GROWS, _HID + 16), jnp.float32)
    zeros80 = jnp.zeros((_GROWS, _FEAT + 16), jnp.float32)

    net, idx3, tapidx, tapw = _run_stage_a(
        p8, p_r, q_r, w8, r1(pr["fc_pos_b"]),
        pr["blk0_W0"], r1(pr["blk0_b0"]),
        pr["blk0_W1"], r1(pr["blk0_b1"]), pr["blk0_Ws"], n)
    idx_xz = idx3[:1]

    for i in (1, 2):
        sums = _scatter1(net, idx_xz, zeros48)[0]
        pooled = _gather_mean(sums, idx_xz[0])
        with_fc = i == 2
        net = _run_block(net, pooled,
                         pr["blk%d_W0" % i], r1(pr["blk%d_b0" % i]),
                         pr["blk%d_W1" % i], r1(pr["blk%d_b1" % i]),
                         pr["blk%d_Ws" % i],
                         pr["fc_c_W"], r1(pr["fc_c_b"]), with_fc)

    s0, s1, s2 = _scatter3(net, idx3, zeros80)
    sums3 = jnp.stack([s0, s1, s2])
    tabs = _run_convs(sums3, convk(pr["conv1_W"]), r1(pr["conv1_b"]),
                      convk(pr["conv2_W"]), r1(pr["conv2_b"]))

    out_cm = _sample(tabs, tapidx, tapw)
    return jnp.transpose(out_cm[:, :m])[None]
